# baseline (speedup 1.0000x reference)
import functools

import jax
import jax.numpy as jnp
from jax import lax
from jax.experimental import pallas as pl
from jax.experimental.pallas import tpu as pltpu
from jax.experimental.pallas import tpu_sc as plsc

_ALPHA = 0.005
_L = 16


@functools.lru_cache(maxsize=None)
def _build(N: int, E: int, B: int):
    NN = N * N
    NC, NS = 2, 16
    BPC = B // NC
    EC = E // NS
    ROWS = N // NS
    STRIPE = ROWS * N
    SCC = 128
    NSC = EC // SCC

    mesh = plsc.VectorSubcoreMesh(core_axis_name="c", subcore_axis_name="s")

    @functools.partial(
        pl.kernel,
        out_type=jax.ShapeDtypeStruct((B * NN,), jnp.float32),
        mesh=mesh,
        scratch_types=[
            pltpu.VMEM((EC,), jnp.int32),
            pltpu.VMEM((EC,), jnp.int32),
            pltpu.VMEM((BPC, EC), jnp.float32),
            pltpu.VMEM((NSC, SCC), jnp.int32),
            pltpu.VMEM((BPC, NSC, SCC), jnp.int32),
            pltpu.VMEM((EC,), jnp.float32),
            pltpu.VMEM((BPC, NSC, SCC), jnp.float32),
            pltpu.SemaphoreType.DMA,
            pltpu.SemaphoreType.DMA,
            pltpu.SemaphoreType.DMA,
            pltpu.SemaphoreType.DMA,
        ],
    )
    def adj_kernel(a_hbm, src_hbm, dst_hbm, g_hbm, out_hbm,
                   src_v, dst_v, g2, off2, offb3, base_v, vals3,
                   sem_c, sem_i, sem_g, sem_s):
        c = lax.axis_index("c")
        s = lax.axis_index("s")

        seg = s * STRIPE
        copies = []
        for bl in range(BPC):
            b = c * BPC + bl
            copies.append(pltpu.async_copy(
                a_hbm.at[pl.ds(seg, STRIPE)],
                out_hbm.at[pl.ds(b * NN + seg, STRIPE)], sem_c))

        e0 = s * EC
        d_src = pltpu.async_copy(src_hbm.at[pl.ds(e0, EC)], src_v, sem_i)
        d_dst = pltpu.async_copy(dst_hbm.at[pl.ds(e0, EC)], dst_v, sem_i)
        d_g = []
        for bl in range(BPC):
            b = c * BPC + bl
            d_g.append(pltpu.async_copy(
                g_hbm.at[pl.ds(b * E + e0, EC)], g2.at[bl], sem_i))

        d_src.wait()
        d_dst.wait()

        def off_chunk(j, carry):
            def off_vec(k, carry2):
                fl = pl.ds(j * SCC + k * _L, _L)
                off2[j, pl.ds(k * _L, _L)] = src_v[fl] * N + dst_v[fl]
                return carry2
            return lax.fori_loop(0, SCC // _L, off_vec, carry)
        lax.fori_loop(0, NSC, off_chunk, 0)

        def g_fire(j, carry):
            pltpu.async_copy(a_hbm.at[off2.at[j]],
                             base_v.at[pl.ds(j * SCC, SCC)], sem_g)
            return carry
        lax.fori_loop(0, NSC, g_fire, 0)

        def g_drain(j, carry):
            pltpu.make_async_copy(a_hbm.at[off2.at[j]],
                                  base_v.at[pl.ds(j * SCC, SCC)], sem_g).wait()
            return carry
        lax.fori_loop(0, NSC, g_drain, 0)

        for bl in range(BPC):
            b = c * BPC + bl
            d_g[bl].wait()

            def val_chunk(j, carry):
                def val_vec(k, carry2):
                    sl = pl.ds(k * _L, _L)
                    fl = pl.ds(j * SCC + k * _L, _L)
                    w = _ALPHA / (1.0 + jnp.exp(-g2[bl, fl]))
                    vals3[bl, j, sl] = base_v[fl] + w
                    offb3[bl, j, sl] = off2[j, sl] + b * NN
                    return carry2
                return lax.fori_loop(0, SCC // _L, val_vec, carry)
            lax.fori_loop(0, NSC, val_chunk, 0)

        for cp in copies:
            cp.wait()
        plsc.subcore_barrier()

        for bl in range(BPC):
            def s_fire(j, carry):
                pltpu.async_copy(vals3.at[bl, j],
                                 out_hbm.at[offb3.at[bl, j]], sem_s)
                return carry
            lax.fori_loop(0, NSC, s_fire, 0)
        for bl in range(BPC):
            def s_drain(j, carry):
                pltpu.make_async_copy(vals3.at[bl, j],
                                      out_hbm.at[offb3.at[bl, j]], sem_s).wait()
                return carry
            lax.fori_loop(0, NSC, s_drain, 0)

    return adj_kernel


def kernel(A_base, edge_index, edge_gates):
    N = A_base.shape[0]
    B, E = edge_gates.shape
    out = _build(N, E, B)(
        A_base.reshape(-1),
        edge_index[0],
        edge_index[1],
        edge_gates.reshape(-1),
    )
    return out.reshape(B, N, N)


# speedup vs baseline: 5.5191x; 5.5191x over previous
import functools

import jax
import jax.numpy as jnp
from jax import lax
from jax.experimental import pallas as pl
from jax.experimental.pallas import tpu as pltpu
from jax.experimental.pallas import tpu_sc as plsc

_ALPHA = 0.005
_L = 16


@functools.lru_cache(maxsize=None)
def _build(N: int, E: int, B: int):
    NN = N * N
    NC, NS = 2, 16
    BPC = B // NC
    EC = E // NS
    ROWS = N // NS
    STRIPE = ROWS * N
    RCH = 16
    CW = RCH * N
    NCH = ROWS // RCH
    SCC = 128
    NSC = EC // SCC

    mesh = plsc.VectorSubcoreMesh(core_axis_name="c", subcore_axis_name="s")

    @functools.partial(
        pl.kernel,
        out_type=jax.ShapeDtypeStruct((B * NN,), jnp.float32),
        mesh=mesh,
        scratch_types=[
            pltpu.VMEM((2, CW), jnp.float32),
            pltpu.VMEM((EC,), jnp.int32),
            pltpu.VMEM((EC,), jnp.int32),
            pltpu.VMEM((BPC, EC), jnp.float32),
            pltpu.VMEM((NSC, SCC), jnp.int32),
            pltpu.VMEM((BPC, NSC, SCC), jnp.int32),
            pltpu.VMEM((EC,), jnp.float32),
            pltpu.VMEM((BPC, NSC, SCC), jnp.float32),
            pltpu.SemaphoreType.DMA,
            pltpu.SemaphoreType.DMA,
            pltpu.SemaphoreType.DMA,
            pltpu.SemaphoreType.DMA,
        ],
    )
    def adj_kernel(a_hbm, src_hbm, dst_hbm, g_hbm, out_hbm,
                   cbuf, src_v, dst_v, g2, off2, offb3, base_v, vals3,
                   sem_c, sem_i, sem_g, sem_s):
        c = lax.axis_index("c")
        s = lax.axis_index("s")

        e0 = s * EC
        d_src = pltpu.async_copy(src_hbm.at[pl.ds(e0, EC)], src_v, sem_i)
        d_dst = pltpu.async_copy(dst_hbm.at[pl.ds(e0, EC)], dst_v, sem_i)
        d_g = []
        for bl in range(BPC):
            b = c * BPC + bl
            d_g.append(pltpu.async_copy(
                g_hbm.at[pl.ds(b * E + e0, EC)], g2.at[bl], sem_i))

        seg0 = s * STRIPE
        rd = [None] * NCH
        wr = [[None] * BPC for _ in range(NCH)]

        def _read(ch):
            return pltpu.async_copy(
                a_hbm.at[pl.ds(seg0 + ch * CW, CW)], cbuf.at[ch % 2], sem_c)

        rd[0] = _read(0)
        for ch in range(NCH):
            if ch + 1 < NCH:
                if ch >= 1:
                    for bl in range(BPC):
                        wr[ch - 1][bl].wait()
                rd[ch + 1] = _read(ch + 1)
            rd[ch].wait()
            for bl in range(BPC):
                b = c * BPC + bl
                wr[ch][bl] = pltpu.async_copy(
                    cbuf.at[ch % 2],
                    out_hbm.at[pl.ds(b * NN + seg0 + ch * CW, CW)], sem_c)

        d_src.wait()
        d_dst.wait()

        def off_chunk(j, carry):
            def off_vec(k, carry2):
                fl = pl.ds(j * SCC + k * _L, _L)
                off2[j, pl.ds(k * _L, _L)] = src_v[fl] * N + dst_v[fl]
                return carry2
            return lax.fori_loop(0, SCC // _L, off_vec, carry)
        lax.fori_loop(0, NSC, off_chunk, 0)

        def g_fire(j, carry):
            pltpu.async_copy(a_hbm.at[off2.at[j]],
                             base_v.at[pl.ds(j * SCC, SCC)], sem_g)
            return carry
        lax.fori_loop(0, NSC, g_fire, 0)

        def g_drain(j, carry):
            pltpu.make_async_copy(a_hbm.at[off2.at[j]],
                                  base_v.at[pl.ds(j * SCC, SCC)], sem_g).wait()
            return carry
        lax.fori_loop(0, NSC, g_drain, 0)

        for bl in range(BPC):
            b = c * BPC + bl
            d_g[bl].wait()

            def val_chunk(j, carry):
                def val_vec(k, carry2):
                    sl = pl.ds(k * _L, _L)
                    fl = pl.ds(j * SCC + k * _L, _L)
                    w = _ALPHA / (1.0 + jnp.exp(-g2[bl, fl]))
                    vals3[bl, j, sl] = base_v[fl] + w
                    offb3[bl, j, sl] = off2[j, sl] + b * NN
                    return carry2
                return lax.fori_loop(0, SCC // _L, val_vec, carry)
            lax.fori_loop(0, NSC, val_chunk, 0)

        for ch in (NCH - 2, NCH - 1):
            for bl in range(BPC):
                wr[ch][bl].wait()
        plsc.subcore_barrier()

        for bl in range(BPC):
            def s_fire(j, carry):
                pltpu.async_copy(vals3.at[bl, j],
                                 out_hbm.at[offb3.at[bl, j]], sem_s)
                return carry
            lax.fori_loop(0, NSC, s_fire, 0)
        for bl in range(BPC):
            def s_drain(j, carry):
                pltpu.make_async_copy(vals3.at[bl, j],
                                      out_hbm.at[offb3.at[bl, j]], sem_s).wait()
                return carry
            lax.fori_loop(0, NSC, s_drain, 0)

    return adj_kernel


def kernel(A_base, edge_index, edge_gates):
    N = A_base.shape[0]
    B, E = edge_gates.shape
    out = _build(N, E, B)(
        A_base.reshape(-1),
        edge_index[0],
        edge_index[1],
        edge_gates.reshape(-1),
    )
    return out.reshape(B, N, N)
